# baseline (speedup 1.0000x reference)
import functools

import jax
import jax.numpy as jnp
from jax import lax
from jax.experimental import pallas as pl
from jax.experimental.pallas import tpu as pltpu
from jax.experimental.pallas import tpu_sc as plsc

N = 10000
E = 160000
D = 256
DH = 128
H = 512
L = 5

N_PAD = 10240
ROWS_PER_TILE = N_PAD // 16
CHUNK = 128
EDGES_PER_TILE = E // 16
WINDOWS = -(-EDGES_PER_TILE // CHUNK)
EPT_PAD = WINDOWS * CHUNK

NBOUND = 31
AUX_BASE = N
DUMP_ROW = N + 32


def _seg_chain(rows_buf, sm_buf, nedges, acc_in):
    def body(j, acc):
        smj = sm_buf[j, :]
        mask = smj != 0.0
        new = []
        for g in range(8):
            r = rows_buf[j, pl.ds(g * 16, 16)]
            v = jnp.where(mask, acc[g] + r, r)
            rows_buf[j, pl.ds(g * 16, 16)] = v
            new.append(v)
        return tuple(new)

    return lax.fori_loop(0, nedges, body, acc_in)


def _sc_msg_body(h_hbm, src3_hbm, dst3_hbm, sm3_hbm, vb_hbm, zeros_hbm,
                 out_hbm, acc, sidx_t, didx_t, rows, smb, mrows, midx,
                 gsem):
    c = lax.axis_index("c")
    s = lax.axis_index("s")
    rb = s * ROWS_PER_TILE
    pltpu.sync_copy(zeros_hbm, acc.at[pl.ds(rb, ROWS_PER_TILE)])
    pltpu.sync_copy(dst3_hbm.at[s], didx_t)
    plsc.subcore_barrier()

    zero16 = jnp.zeros((16,), jnp.float32)
    acc0 = (zero16,) * 8

    def step(w, carry):
        pltpu.sync_copy(src3_hbm.at[c * 16 + s, w], sidx_t.at[0])
        pltpu.sync_copy(sm3_hbm.at[s, pl.ds(w * CHUNK, CHUNK)], smb)
        pltpu.async_copy(h_hbm.at[sidx_t.at[0]], rows, gsem).wait()
        carry = _seg_chain(rows, smb, CHUNK, carry)
        pltpu.sync_copy(rows, acc.at[didx_t.at[w]])
        return carry

    lax.fori_loop(0, WINDOWS, step, acc0)
    plsc.subcore_barrier()

    @pl.when(s == 0)
    def _():
        pltpu.sync_copy(vb_hbm, midx)
        pltpu.sync_copy(acc.at[pl.ds(AUX_BASE, 32)], mrows)
        pltpu.sync_copy(mrows, acc.at[midx.at[0]], add=True)

    plsc.subcore_barrier()
    pltpu.sync_copy(acc.at[pl.ds(rb, ROWS_PER_TILE)],
                    out_hbm.at[pl.ds(c * N_PAD + rb, ROWS_PER_TILE)])


@functools.cache
def _make_sc_msg():
    return pl.kernel(
        _sc_msg_body,
        out_type=jax.ShapeDtypeStruct((2 * N_PAD, DH), jnp.float32),
        mesh=plsc.VectorSubcoreMesh(core_axis_name="c", subcore_axis_name="s"),
        scratch_types=[
            pltpu.VMEM_SHARED((N_PAD, DH), jnp.float32),
            pltpu.VMEM((1, CHUNK), jnp.int32),
            pltpu.VMEM((WINDOWS, CHUNK), jnp.int32),
            pltpu.VMEM((CHUNK, DH), jnp.float32),
            pltpu.VMEM((CHUNK, 16), jnp.float32),
            pltpu.VMEM((32, DH), jnp.float32),
            pltpu.VMEM((1, 32), jnp.int32),
            pltpu.SemaphoreType.DMA,
        ],
    )


def _sc_msg(h_split, src3, dst3, sm3, vb, zeros):
    return _make_sc_msg()(h_split, src3, dst3, sm3, vb, zeros)


def _prep_edges(edge_index):
    src = edge_index[0]
    dst = edge_index[1]
    order = jnp.argsort(dst, stable=True)
    ss = src[order]
    ds = dst[order]
    csize = E // 32
    starts = ds[jnp.arange(1, 32) * csize]
    befores = ds[jnp.arange(1, 32) * csize - 1]
    strad = starts == befores
    sval = jnp.where(strad, starts, -1)
    cid = jnp.arange(E, dtype=jnp.int32) // csize
    svals_full = jnp.concatenate([jnp.array([-1], jnp.int32), sval])
    edge_sval = svals_full[cid]
    ds_remap = jnp.where(ds == edge_sval,
                         AUX_BASE + cid - 1, ds).astype(jnp.int32)
    prev = jnp.concatenate([jnp.full((1,), -2, jnp.int32), ds_remap[:-1]])
    at_chunk_start = (jnp.arange(E, dtype=jnp.int32) % csize) == 0
    same = (ds_remap == prev) & (~at_chunk_start)
    same16 = jnp.broadcast_to(
        same.astype(jnp.float32)[:, None], (E, 16))
    nxt = jnp.concatenate([ds_remap[1:], jnp.full((1,), -2, jnp.int32)])
    at_chunk_end = ((jnp.arange(E, dtype=jnp.int32) + 1) % csize) == 0
    last = (ds_remap != nxt) | at_chunk_end
    didx2 = jnp.where(last, ds_remap, DUMP_ROW).astype(jnp.int32)
    vb = jnp.where(strad, starts, DUMP_ROW).astype(jnp.int32)
    vb = jnp.concatenate([vb, jnp.full((1,), DUMP_ROW, jnp.int32)])
    src2 = jnp.concatenate([ss, ss + N_PAD])

    pad_t = EPT_PAD - EDGES_PER_TILE
    src3 = jnp.pad(src2.reshape(32, EDGES_PER_TILE),
                   ((0, 0), (0, pad_t))).reshape(32, WINDOWS, CHUNK)
    dst3 = jnp.pad(didx2.reshape(16, EDGES_PER_TILE),
                   ((0, 0), (0, pad_t)),
                   constant_values=DUMP_ROW).reshape(16, WINDOWS, CHUNK)
    sm3 = jnp.pad(same16.reshape(16, EDGES_PER_TILE, 16),
                  ((0, 0), (0, pad_t), (0, 0)))
    return src3, dst3, sm3, vb.reshape(1, 32)


def _split_layout(h):
    out = jnp.zeros((2 * N_PAD, DH), jnp.float32)
    return out.at[:N].set(h[:, :DH]).at[N_PAD:N_PAD + N].set(h[:, DH:])


def _unsplit(ms):
    return jnp.concatenate([ms[:N], ms[N_PAD:N_PAD + N]], axis=1)


def _m1_body(eps_ref, h_ref, msg_ref, w1_ref, b1_ref, u_ref):
    agg = (1.0 + eps_ref[0, 0]) * h_ref[...] + msg_ref[...]
    u_ref[...] = jax.nn.relu(
        jnp.dot(agg, w1_ref[...], preferred_element_type=jnp.float32)
        + b1_ref[...])


_m1 = pl.pallas_call(
    _m1_body,
    in_specs=[
        pl.BlockSpec(memory_space=pltpu.SMEM),
        pl.BlockSpec((N, D), lambda: (0, 0)),
        pl.BlockSpec((N, D), lambda: (0, 0)),
        pl.BlockSpec((D, H), lambda: (0, 0)),
        pl.BlockSpec((1, H), lambda: (0, 0)),
    ],
    out_specs=pl.BlockSpec((N, H), lambda: (0, 0)),
    out_shape=jax.ShapeDtypeStruct((N, H), jnp.float32),
)


def _m2_body(u_ref, w2_ref, b2_ref, t_ref):
    t_ref[...] = (jnp.dot(u_ref[...], w2_ref[...],
                          preferred_element_type=jnp.float32) + b2_ref[...])


_m2 = pl.pallas_call(
    _m2_body,
    in_specs=[
        pl.BlockSpec((N, H), lambda: (0, 0)),
        pl.BlockSpec((H, D), lambda: (0, 0)),
        pl.BlockSpec((1, D), lambda: (0, 0)),
    ],
    out_specs=pl.BlockSpec((N, D), lambda: (0, 0)),
    out_shape=jax.ShapeDtypeStruct((N, D), jnp.float32),
)


_INV_N = float(jnp.float32(1.0 / N))


def _col_reduce(read_tile):
    halves = []
    for hbase in (0, 625):
        def step(r, acc, hbase=hbase):
            return read_tile(hbase + r) + acc
        acc = lax.fori_loop(1, 625, step, read_tile(hbase))
        a = acc[0:4] + acc[4:8]
        b = a[0:2] + a[2:4]
        halves.append(b[0:1] + b[1:2])
    return halves[0] + halves[1]


def _stats_body(t_ref, mean_ref, var_ref, dev_ref):
    s = _col_reduce(lambda r: t_ref[pl.ds(r * 8, 8), :])
    mean = s * _INV_N
    mean_ref[...] = mean
    d = t_ref[...] - mean
    dev_ref[...] = d * d
    sv = _col_reduce(lambda r: dev_ref[pl.ds(r * 8, 8), :])
    var_ref[...] = sv * _INV_N


_stats = pl.pallas_call(
    _stats_body,
    in_specs=[pl.BlockSpec((N, D), lambda: (0, 0))],
    out_specs=[pl.BlockSpec((1, D), lambda: (0, 0)),
               pl.BlockSpec((1, D), lambda: (0, 0))],
    out_shape=[jax.ShapeDtypeStruct((1, D), jnp.float32),
               jax.ShapeDtypeStruct((1, D), jnp.float32)],
    scratch_shapes=[pltpu.VMEM((N, D), jnp.float32)],
)


def _norm_body(t_ref, m_ref, v_ref, g_ref, b_ref, h_ref, *, relu):
    hv = ((t_ref[...] - m_ref[...]) / jnp.sqrt(v_ref[...] + 1e-5)
          * g_ref[...] + b_ref[...])
    if relu:
        hv = jax.nn.relu(hv)
    h_ref[...] = hv


def _make_norm(relu):
    return pl.pallas_call(
        functools.partial(_norm_body, relu=relu),
        in_specs=[
            pl.BlockSpec((N, D), lambda: (0, 0)),
            pl.BlockSpec((1, D), lambda: (0, 0)),
            pl.BlockSpec((1, D), lambda: (0, 0)),
            pl.BlockSpec((1, D), lambda: (0, 0)),
            pl.BlockSpec((1, D), lambda: (0, 0)),
        ],
        out_specs=pl.BlockSpec((N, D), lambda: (0, 0)),
        out_shape=jax.ShapeDtypeStruct((N, D), jnp.float32),
    )


_norm_relu = _make_norm(True)
_norm_last = _make_norm(False)


def _pool_body(h_ref, g_ref):
    s = _col_reduce(lambda r: h_ref[pl.ds(r * 8, 8), :])
    g_ref[...] = s * _INV_N


_pool = pl.pallas_call(
    _pool_body,
    in_specs=[pl.BlockSpec((N, D), lambda: (0, 0))],
    out_specs=pl.BlockSpec((1, D), lambda: (0, 0)),
    out_shape=jax.ShapeDtypeStruct((1, D), jnp.float32),
)


def kernel(x, edge_index, W1, b1, W2, b2, eps, gamma, beta):
    src3, dst3, sm3, vb = _prep_edges(edge_index)
    zeros = jnp.zeros((ROWS_PER_TILE, DH), jnp.float32)

    h = x
    for l in range(L):
        msg = _unsplit(_sc_msg(_split_layout(h), src3, dst3, sm3,
                               vb, zeros))
        u = _m1(eps[l].reshape(1, 1), h, msg, W1[l], b1[l].reshape(1, H))
        t = _m2(u, W2[l], b2[l].reshape(1, D))
        mean, var = _stats(t)
        g = gamma[l].reshape(1, D)
        bt = beta[l].reshape(1, D)
        if l < L - 1:
            h = _norm_relu(t, mean, var, g, bt)
        else:
            h = _norm_last(t, mean, var, g, bt)

    h_graph = _pool(h)
    return (h_graph, h)


# speedup vs baseline: 1.1526x; 1.1526x over previous
import functools

import jax
import jax.numpy as jnp
from jax import lax
from jax.experimental import pallas as pl
from jax.experimental.pallas import tpu as pltpu
from jax.experimental.pallas import tpu_sc as plsc

N = 10000
E = 160000
D = 256
DH = 128
H = 512
L = 5

N_PAD = 10240
ROWS_PER_TILE = N_PAD // 16
CHUNK = 128
EDGES_PER_TILE = E // 16
FULL_CHUNKS = EDGES_PER_TILE // CHUNK
TAIL = EDGES_PER_TILE - FULL_CHUNKS * CHUNK

NBOUND = 31
AUX_BASE = N
DUMP_ROW = N + 32


def _seg_chain(rows_buf, sm_buf, nedges, acc_in):
    def body(j, acc):
        smj = sm_buf[j, :]
        mask = smj != 0.0
        new = []
        for g in range(8):
            r = rows_buf[j, pl.ds(g * 16, 16)]
            v = jnp.where(mask, acc[g] + r, r)
            rows_buf[j, pl.ds(g * 16, 16)] = v
            new.append(v)
        return tuple(new)

    return lax.fori_loop(0, nedges, body, acc_in)


def _sc_msg_body(h_hbm, src2_hbm, dst_hbm, sm_hbm, vb_hbm, zeros_hbm,
                 out_hbm, acc, sidx, didx, rows, smb, sidx16, didx16,
                 rows16, smb16, mrows, midx, gsem):
    c = lax.axis_index("c")
    s = lax.axis_index("s")
    rb = s * ROWS_PER_TILE
    pltpu.sync_copy(zeros_hbm, acc.at[pl.ds(rb, ROWS_PER_TILE)])
    plsc.subcore_barrier()

    ebase = s * EDGES_PER_TILE
    zero16 = jnp.zeros((16,), jnp.float32)
    acc0 = (zero16,) * 8

    def step(j, carry):
        off = ebase + j * CHUNK
        pltpu.sync_copy(src2_hbm.at[pl.ds(c * E + off, CHUNK)], sidx.at[0])
        pltpu.sync_copy(dst_hbm.at[pl.ds(off, CHUNK)], didx.at[0])
        pltpu.sync_copy(sm_hbm.at[pl.ds(off, CHUNK)], smb)
        pltpu.async_copy(h_hbm.at[sidx.at[0]], rows, gsem).wait()
        carry = _seg_chain(rows, smb, CHUNK, carry)
        pltpu.sync_copy(rows, acc.at[didx.at[0]])
        return carry

    accv = lax.fori_loop(0, FULL_CHUNKS, step, acc0)
    toff = ebase + FULL_CHUNKS * CHUNK
    pltpu.sync_copy(src2_hbm.at[pl.ds(c * E + toff, TAIL)], sidx16.at[0])
    pltpu.sync_copy(dst_hbm.at[pl.ds(toff, TAIL)], didx16.at[0])
    pltpu.sync_copy(sm_hbm.at[pl.ds(toff, TAIL)], smb16)
    pltpu.async_copy(h_hbm.at[sidx16.at[0]], rows16, gsem).wait()
    _seg_chain(rows16, smb16, TAIL, accv)
    pltpu.sync_copy(rows16, acc.at[didx16.at[0]])
    plsc.subcore_barrier()

    @pl.when(s == 0)
    def _():
        pltpu.sync_copy(vb_hbm, midx)
        pltpu.sync_copy(acc.at[pl.ds(AUX_BASE, 32)], mrows)
        pltpu.sync_copy(mrows, acc.at[midx.at[0]], add=True)

    plsc.subcore_barrier()
    pltpu.sync_copy(acc.at[pl.ds(rb, ROWS_PER_TILE)],
                    out_hbm.at[pl.ds(c * N_PAD + rb, ROWS_PER_TILE)])


@functools.cache
def _make_sc_msg():
    return pl.kernel(
        _sc_msg_body,
        out_type=jax.ShapeDtypeStruct((2 * N_PAD, DH), jnp.float32),
        mesh=plsc.VectorSubcoreMesh(core_axis_name="c", subcore_axis_name="s"),
        scratch_types=[
            pltpu.VMEM_SHARED((N_PAD, DH), jnp.float32),
            pltpu.VMEM((1, CHUNK), jnp.int32),
            pltpu.VMEM((1, CHUNK), jnp.int32),
            pltpu.VMEM((CHUNK, DH), jnp.float32),
            pltpu.VMEM((CHUNK, 16), jnp.float32),
            pltpu.VMEM((1, TAIL), jnp.int32),
            pltpu.VMEM((1, TAIL), jnp.int32),
            pltpu.VMEM((TAIL, DH), jnp.float32),
            pltpu.VMEM((TAIL, 16), jnp.float32),
            pltpu.VMEM((32, DH), jnp.float32),
            pltpu.VMEM((1, 32), jnp.int32),
            pltpu.SemaphoreType.DMA,
        ],
    )


def _sc_msg(h_split, src2, didx2, same16, vb, zeros):
    return _make_sc_msg()(h_split, src2, didx2, same16, vb, zeros)


def _prep_edges(edge_index):
    src = edge_index[0]
    dst = edge_index[1]
    order = jnp.argsort(dst, stable=True)
    ss = src[order]
    ds = dst[order]
    csize = E // 32
    starts = ds[jnp.arange(1, 32) * csize]
    befores = ds[jnp.arange(1, 32) * csize - 1]
    strad = starts == befores
    sval = jnp.where(strad, starts, -1)
    cid = jnp.arange(E, dtype=jnp.int32) // csize
    svals_full = jnp.concatenate([jnp.array([-1], jnp.int32), sval])
    edge_sval = svals_full[cid]
    ds_remap = jnp.where(ds == edge_sval,
                         AUX_BASE + cid - 1, ds).astype(jnp.int32)
    prev = jnp.concatenate([jnp.full((1,), -2, jnp.int32), ds_remap[:-1]])
    at_chunk_start = (jnp.arange(E, dtype=jnp.int32) % csize) == 0
    same = (ds_remap == prev) & (~at_chunk_start)
    same16 = jnp.broadcast_to(
        same.astype(jnp.float32)[:, None], (E, 16))
    nxt = jnp.concatenate([ds_remap[1:], jnp.full((1,), -2, jnp.int32)])
    at_chunk_end = ((jnp.arange(E, dtype=jnp.int32) + 1) % csize) == 0
    last = (ds_remap != nxt) | at_chunk_end
    didx2 = jnp.where(last, ds_remap, DUMP_ROW).astype(jnp.int32)
    vb = jnp.where(strad, starts, DUMP_ROW).astype(jnp.int32)
    vb = jnp.concatenate([vb, jnp.full((1,), DUMP_ROW, jnp.int32)])
    src2 = jnp.concatenate([ss, ss + N_PAD])
    return src2, didx2, same16, vb.reshape(1, 32)


def _split_layout(h):
    out = jnp.zeros((2 * N_PAD, DH), jnp.float32)
    return out.at[:N].set(h[:, :DH]).at[N_PAD:N_PAD + N].set(h[:, DH:])


def _unsplit(ms):
    return jnp.concatenate([ms[:N], ms[N_PAD:N_PAD + N]], axis=1)


def _m1_body(eps_ref, h_ref, msg_ref, w1_ref, b1_ref, u_ref):
    agg = (1.0 + eps_ref[0, 0]) * h_ref[...] + msg_ref[...]
    u_ref[...] = jax.nn.relu(
        jnp.dot(agg, w1_ref[...], preferred_element_type=jnp.float32)
        + b1_ref[...])


_m1 = pl.pallas_call(
    _m1_body,
    in_specs=[
        pl.BlockSpec(memory_space=pltpu.SMEM),
        pl.BlockSpec((N, D), lambda: (0, 0)),
        pl.BlockSpec((N, D), lambda: (0, 0)),
        pl.BlockSpec((D, H), lambda: (0, 0)),
        pl.BlockSpec((1, H), lambda: (0, 0)),
    ],
    out_specs=pl.BlockSpec((N, H), lambda: (0, 0)),
    out_shape=jax.ShapeDtypeStruct((N, H), jnp.float32),
)


def _m2_body(u_ref, w2_ref, b2_ref, t_ref):
    t_ref[...] = (jnp.dot(u_ref[...], w2_ref[...],
                          preferred_element_type=jnp.float32) + b2_ref[...])


_m2 = pl.pallas_call(
    _m2_body,
    in_specs=[
        pl.BlockSpec((N, H), lambda: (0, 0)),
        pl.BlockSpec((H, D), lambda: (0, 0)),
        pl.BlockSpec((1, D), lambda: (0, 0)),
    ],
    out_specs=pl.BlockSpec((N, D), lambda: (0, 0)),
    out_shape=jax.ShapeDtypeStruct((N, D), jnp.float32),
)


_INV_N = float(jnp.float32(1.0 / N))


def _col_reduce(read_tile):
    halves = []
    for hbase in (0, 625):
        def step(r, acc, hbase=hbase):
            return read_tile(hbase + r) + acc
        acc = lax.fori_loop(1, 625, step, read_tile(hbase))
        a = acc[0:4] + acc[4:8]
        b = a[0:2] + a[2:4]
        halves.append(b[0:1] + b[1:2])
    return halves[0] + halves[1]


def _stats_body(t_ref, mean_ref, var_ref, dev_ref):
    s = _col_reduce(lambda r: t_ref[pl.ds(r * 8, 8), :])
    mean = s * _INV_N
    mean_ref[...] = mean
    d = t_ref[...] - mean
    dev_ref[...] = d * d
    sv = _col_reduce(lambda r: dev_ref[pl.ds(r * 8, 8), :])
    var_ref[...] = sv * _INV_N


_stats = pl.pallas_call(
    _stats_body,
    in_specs=[pl.BlockSpec((N, D), lambda: (0, 0))],
    out_specs=[pl.BlockSpec((1, D), lambda: (0, 0)),
               pl.BlockSpec((1, D), lambda: (0, 0))],
    out_shape=[jax.ShapeDtypeStruct((1, D), jnp.float32),
               jax.ShapeDtypeStruct((1, D), jnp.float32)],
    scratch_shapes=[pltpu.VMEM((N, D), jnp.float32)],
)


def _norm_body(t_ref, m_ref, v_ref, g_ref, b_ref, h_ref, *, relu):
    hv = ((t_ref[...] - m_ref[...]) / jnp.sqrt(v_ref[...] + 1e-5)
          * g_ref[...] + b_ref[...])
    if relu:
        hv = jax.nn.relu(hv)
    h_ref[...] = hv


def _make_norm(relu):
    return pl.pallas_call(
        functools.partial(_norm_body, relu=relu),
        in_specs=[
            pl.BlockSpec((N, D), lambda: (0, 0)),
            pl.BlockSpec((1, D), lambda: (0, 0)),
            pl.BlockSpec((1, D), lambda: (0, 0)),
            pl.BlockSpec((1, D), lambda: (0, 0)),
            pl.BlockSpec((1, D), lambda: (0, 0)),
        ],
        out_specs=pl.BlockSpec((N, D), lambda: (0, 0)),
        out_shape=jax.ShapeDtypeStruct((N, D), jnp.float32),
    )


_norm_relu = _make_norm(True)
_norm_last = _make_norm(False)


def _pool_body(h_ref, g_ref):
    s = _col_reduce(lambda r: h_ref[pl.ds(r * 8, 8), :])
    g_ref[...] = s * _INV_N


_pool = pl.pallas_call(
    _pool_body,
    in_specs=[pl.BlockSpec((N, D), lambda: (0, 0))],
    out_specs=pl.BlockSpec((1, D), lambda: (0, 0)),
    out_shape=jax.ShapeDtypeStruct((1, D), jnp.float32),
)


def kernel(x, edge_index, W1, b1, W2, b2, eps, gamma, beta):
    src2, didx2, same16, vb = _prep_edges(edge_index)
    zeros = jnp.zeros((ROWS_PER_TILE, DH), jnp.float32)

    h = x
    for l in range(L):
        msg = _unsplit(_sc_msg(_split_layout(h), src2, didx2, same16,
                               vb, zeros))
        u = _m1(eps[l].reshape(1, 1), h, msg, W1[l], b1[l].reshape(1, H))
        t = _m2(u, W2[l], b2[l].reshape(1, D))
        mean, var = _stats(t)
        g = gamma[l].reshape(1, D)
        bt = beta[l].reshape(1, D)
        if l < L - 1:
            h = _norm_relu(t, mean, var, g, bt)
        else:
            h = _norm_last(t, mean, var, g, bt)

    h_graph = _pool(h)
    return (h_graph, h)
